# baseline (speedup 1.0000x reference)
import functools

import jax
import jax.numpy as jnp
from jax import lax
from jax.experimental import pallas as pl
from jax.experimental.pallas import tpu as pltpu
from jax.experimental.pallas import tpu_sc as plsc

N = 10000
H = 256
E = 160000
BLK = 2000
F32 = jnp.float32


def _elu(x):
    return jnp.where(x > 0, x, jnp.exp(x) - 1.0)


def _split_outs(p, n_tbl, tbl_refs, self_ref):
    for j in range(n_tbl):
        tbl_refs[j][0] = p[:, j * 256:j * 256 + 128]
        tbl_refs[j][1] = p[:, j * 256 + 128:(j + 1) * 256]
    self_ref[...] = p[:, n_tbl * 256:]


def _enc_proj_kernel(n_tbl, x_ref, wenc_ref, benc_ref, emb_ref, cw_ref, *outs):
    t = jnp.dot(x_ref[...], wenc_ref[...], preferred_element_type=F32)
    t = t + benc_ref[...] + emb_ref[...]
    p = jnp.dot(t, cw_ref[...], preferred_element_type=F32)
    _split_outs(p, n_tbl, outs[:n_tbl], outs[n_tbl])


def _plain_proj_kernel(n_tbl, x_ref, cw_ref, *outs):
    p = jnp.dot(x_ref[...], cw_ref[...], preferred_element_type=F32)
    _split_outs(p, n_tbl, outs[:n_tbl], outs[n_tbl])


def _combine_proj_kernel(n_agg, n_tbl, *refs):
    aggs = refs[:n_agg]
    selfp_ref, bias_ref, cw_ref = refs[n_agg], refs[n_agg + 1], refs[n_agg + 2]
    outs = refs[n_agg + 3:]
    acc = selfp_ref[...] + bias_ref[...]
    for a in aggs:
        acc = acc + jnp.concatenate([a[0], a[1]], axis=1)
    t = _elu(acc)
    p = jnp.dot(t, cw_ref[...], preferred_element_type=F32)
    _split_outs(p, n_tbl, outs[:n_tbl], outs[n_tbl])


def _combine_final_kernel(n_agg, *refs):
    aggs = refs[:n_agg]
    selfp_ref, bias_ref, out_ref = refs[n_agg], refs[n_agg + 1], refs[n_agg + 2]
    acc = selfp_ref[...] + bias_ref[...]
    for a in aggs:
        acc = acc + jnp.concatenate([a[0], a[1]], axis=1)
    out_ref[...] = _elu(acc)


def _tbl_spec():
    return pl.BlockSpec((2, BLK, 128), lambda i: (0, i, 0))


def _row_spec(width):
    return pl.BlockSpec((BLK, width), lambda i: (i, 0))


def _full_spec(shape):
    return pl.BlockSpec(shape, lambda i: tuple(0 for _ in shape))


def _proj_outs(n_tbl):
    shapes = [jax.ShapeDtypeStruct((2, N, 128), F32) for _ in range(n_tbl)]
    shapes.append(jax.ShapeDtypeStruct((N, H), F32))
    specs = [_tbl_spec() for _ in range(n_tbl)] + [_row_spec(H)]
    return shapes, specs


def _enc_proj(x, wenc, benc, emb, cw, n_tbl):
    din = x.shape[1]
    out_shapes, out_specs = _proj_outs(n_tbl)
    return pl.pallas_call(
        functools.partial(_enc_proj_kernel, n_tbl),
        grid=(N // BLK,),
        in_specs=[_row_spec(din), _full_spec((din, H)), _full_spec((1, H)),
                  _row_spec(H), _full_spec(cw.shape)],
        out_specs=out_specs,
        out_shape=out_shapes,
    )(x, wenc, benc.reshape(1, H), emb, cw)


def _plain_proj(x, cw, n_tbl):
    out_shapes, out_specs = _proj_outs(n_tbl)
    return pl.pallas_call(
        functools.partial(_plain_proj_kernel, n_tbl),
        grid=(N // BLK,),
        in_specs=[_row_spec(H), _full_spec(cw.shape)],
        out_specs=out_specs,
        out_shape=out_shapes,
    )(x, cw)


def _combine_proj(aggs, selfp, bias, cw, n_tbl):
    n_agg = len(aggs)
    out_shapes, out_specs = _proj_outs(n_tbl)
    in_specs = ([_tbl_spec() for _ in range(n_agg)]
                + [_row_spec(H), _full_spec((1, H)), _full_spec(cw.shape)])
    return pl.pallas_call(
        functools.partial(_combine_proj_kernel, n_agg, n_tbl),
        grid=(N // BLK,),
        in_specs=in_specs,
        out_specs=out_specs,
        out_shape=out_shapes,
    )(*aggs, selfp, bias.reshape(1, H), cw)


def _combine_final(aggs, selfp, bias):
    n_agg = len(aggs)
    in_specs = ([_tbl_spec() for _ in range(n_agg)]
                + [_row_spec(H), _full_spec((1, H))])
    return pl.pallas_call(
        functools.partial(_combine_final_kernel, n_agg),
        grid=(N // BLK,),
        in_specs=in_specs,
        out_specs=_row_spec(H),
        out_shape=jax.ShapeDtypeStruct((N, H), F32),
    )(*aggs, selfp, bias.reshape(1, H))


def _seg_mean_tables(tables, srcs, dsts, invs):
    outs = []
    for tbl, src, dst, inv in zip(tables, srcs, dsts, invs):
        flat = tbl.reshape(2 * N, 128)
        res = []
        for h in range(2):
            msg = jnp.take(flat, src + h * N, axis=0)
            agg = jax.ops.segment_sum(msg, dst, num_segments=N)
            res.append(agg * inv[:, None])
        outs.append(jnp.stack(res, axis=0))
    return outs


def kernel(x_sub, x_module, params, node_id_sub, node_id_bay, node_id_module,
           edge_index_sub_bay, edge_index_bay_module):
    p = params
    sb0, sb1 = edge_index_sub_bay[0], edge_index_sub_bay[1]
    bm0, bm1 = edge_index_bay_module[0], edge_index_bay_module[1]

    srcs = [sb0, bm1, sb1, bm0]
    dsts = [sb1, bm0, sb0, bm1]

    ones = jnp.ones((E,), F32)
    invs = [1.0 / jnp.clip(jax.ops.segment_sum(ones, d, num_segments=N), 1.0)
            for d in dsts]

    def cw_sub(l):
        return jnp.concatenate([p['l%d_sb_Wl' % l], p['l%d_bs_Wr' % l]], axis=1)

    def cw_bay(l):
        return jnp.concatenate([p['l%d_bs_Wl' % l], p['l%d_bm_Wl' % l],
                                p['l%d_sb_Wr' % l] + p['l%d_mb_Wr' % l]], axis=1)

    def cw_mod(l):
        return jnp.concatenate([p['l%d_mb_Wl' % l], p['l%d_bm_Wr' % l]], axis=1)

    tbl_sb, self_sub = _enc_proj(x_sub, p['W_lin_sub'], p['b_lin_sub'],
                                 p['emb_sub'], cw_sub(1), 1)
    tbl_bs, tbl_bm, self_bay = _plain_proj(p['emb_bay'], cw_bay(1), 2)
    tbl_mb, self_mod = _enc_proj(x_module, p['W_lin_module'], p['b_lin_module'],
                                 p['emb_module'], cw_mod(1), 1)

    for l in (1, 2, 3):
        a_sb, a_mb, a_bs, a_bm = _seg_mean_tables(
            [tbl_sb, tbl_mb, tbl_bs, tbl_bm], srcs, dsts, invs)
        bl_sub = p['l%d_bs_bl' % l]
        bl_bay = p['l%d_sb_bl' % l] + p['l%d_mb_bl' % l]
        bl_mod = p['l%d_bm_bl' % l]
        if l < 3:
            tbl_sb, self_sub = _combine_proj([a_bs], self_sub, bl_sub,
                                             cw_sub(l + 1), 1)
            tbl_bs, tbl_bm, self_bay = _combine_proj([a_sb, a_mb], self_bay,
                                                     bl_bay, cw_bay(l + 1), 2)
            tbl_mb, self_mod = _combine_proj([a_bm], self_mod, bl_mod,
                                             cw_mod(l + 1), 1)
        else:
            out_sub = _combine_final([a_bs], self_sub, bl_sub)
            out_bay = _combine_final([a_sb, a_mb], self_bay, bl_bay)
            out_mod = _combine_final([a_bm], self_mod, bl_mod)

    return (out_sub, out_bay, out_mod)


# speedup vs baseline: 2.0077x; 2.0077x over previous
import functools

import jax
import jax.numpy as jnp
from jax import lax
from jax.experimental import pallas as pl
from jax.experimental.pallas import tpu as pltpu
from jax.experimental.pallas import tpu_sc as plsc

N = 10000
H = 256
E = 160000
BLK = 2000
F32 = jnp.float32


def _elu(x):
    return jnp.where(x > 0, x, jnp.exp(x) - 1.0)


def _split_outs(p, n_tbl, tbl_refs, self_ref):
    for j in range(n_tbl):
        tbl_refs[j][0] = p[:, j * 256:j * 256 + 128]
        tbl_refs[j][1] = p[:, j * 256 + 128:(j + 1) * 256]
    self_ref[...] = p[:, n_tbl * 256:]


def _enc_proj_kernel(n_tbl, x_ref, wenc_ref, benc_ref, emb_ref, cw_ref, *outs):
    t = jnp.dot(x_ref[...], wenc_ref[...], preferred_element_type=F32)
    t = t + benc_ref[...] + emb_ref[...]
    p = jnp.dot(t, cw_ref[...], preferred_element_type=F32)
    _split_outs(p, n_tbl, outs[:n_tbl], outs[n_tbl])


def _plain_proj_kernel(n_tbl, x_ref, cw_ref, *outs):
    p = jnp.dot(x_ref[...], cw_ref[...], preferred_element_type=F32)
    _split_outs(p, n_tbl, outs[:n_tbl], outs[n_tbl])


def _combine_proj_kernel(n_agg, n_tbl, *refs):
    aggs = refs[:n_agg]
    selfp_ref, bias_ref, cw_ref = refs[n_agg], refs[n_agg + 1], refs[n_agg + 2]
    outs = refs[n_agg + 3:]
    acc = selfp_ref[...] + bias_ref[...]
    for a in aggs:
        acc = acc + jnp.concatenate([a[0], a[1]], axis=1)
    t = _elu(acc)
    p = jnp.dot(t, cw_ref[...], preferred_element_type=F32)
    _split_outs(p, n_tbl, outs[:n_tbl], outs[n_tbl])


def _combine_final_kernel(n_agg, *refs):
    aggs = refs[:n_agg]
    selfp_ref, bias_ref, out_ref = refs[n_agg], refs[n_agg + 1], refs[n_agg + 2]
    acc = selfp_ref[...] + bias_ref[...]
    for a in aggs:
        acc = acc + jnp.concatenate([a[0], a[1]], axis=1)
    out_ref[...] = _elu(acc)


def _tbl_spec():
    return pl.BlockSpec((2, BLK, 128), lambda i: (0, i, 0))


def _row_spec(width):
    return pl.BlockSpec((BLK, width), lambda i: (i, 0))


def _full_spec(shape):
    return pl.BlockSpec(shape, lambda i: tuple(0 for _ in shape))


def _proj_outs(n_tbl):
    shapes = [jax.ShapeDtypeStruct((2, N, 128), F32) for _ in range(n_tbl)]
    shapes.append(jax.ShapeDtypeStruct((N, H), F32))
    specs = [_tbl_spec() for _ in range(n_tbl)] + [_row_spec(H)]
    return shapes, specs


def _enc_proj(x, wenc, benc, emb, cw, n_tbl):
    din = x.shape[1]
    out_shapes, out_specs = _proj_outs(n_tbl)
    return pl.pallas_call(
        functools.partial(_enc_proj_kernel, n_tbl),
        grid=(N // BLK,),
        in_specs=[_row_spec(din), _full_spec((din, H)), _full_spec((1, H)),
                  _row_spec(H), _full_spec(cw.shape)],
        out_specs=out_specs,
        out_shape=out_shapes,
    )(x, wenc, benc.reshape(1, H), emb, cw)


def _plain_proj(x, cw, n_tbl):
    out_shapes, out_specs = _proj_outs(n_tbl)
    return pl.pallas_call(
        functools.partial(_plain_proj_kernel, n_tbl),
        grid=(N // BLK,),
        in_specs=[_row_spec(H), _full_spec(cw.shape)],
        out_specs=out_specs,
        out_shape=out_shapes,
    )(x, cw)


def _combine_proj(aggs, selfp, bias, cw, n_tbl):
    n_agg = len(aggs)
    out_shapes, out_specs = _proj_outs(n_tbl)
    in_specs = ([_tbl_spec() for _ in range(n_agg)]
                + [_row_spec(H), _full_spec((1, H)), _full_spec(cw.shape)])
    return pl.pallas_call(
        functools.partial(_combine_proj_kernel, n_agg, n_tbl),
        grid=(N // BLK,),
        in_specs=in_specs,
        out_specs=out_specs,
        out_shape=out_shapes,
    )(*aggs, selfp, bias.reshape(1, H), cw)


def _combine_final(aggs, selfp, bias):
    n_agg = len(aggs)
    in_specs = ([_tbl_spec() for _ in range(n_agg)]
                + [_row_spec(H), _full_spec((1, H))])
    return pl.pallas_call(
        functools.partial(_combine_final_kernel, n_agg),
        grid=(N // BLK,),
        in_specs=in_specs,
        out_specs=_row_spec(H),
        out_shape=jax.ShapeDtypeStruct((N, H), F32),
    )(*aggs, selfp, bias.reshape(1, H))


NSUB = 16
NP = 10240
RPT = NP // NSUB
TRASH = N
LB = 1024
CAPB = 160
SCH = 2000
CH = 64
KB = 4
FCH = 16
NFCH = RPT // FCH
_SC_PARAMS = pltpu.CompilerParams(needs_layout_passes=False)
_MESH = plsc.VectorSubcoreMesh(core_axis_name="c", subcore_axis_name="s",
                               num_cores=2, num_subcores=16)


def _sc_prep_body(*refs):
    it = iter(refs)
    srcs = [next(it) for _ in range(4)]
    dsts = [next(it) for _ in range(4)]
    ones_h = next(it)
    z16_h = next(it)
    slist = next(it)
    dlist = next(it)
    nb_out = next(it)
    inv_out = next(it)
    cacc = next(it)
    sbuf = next(it)
    dbuf = next(it)
    cbs = next(it)
    cbd = next(it)
    dbuf2 = next(it)
    ones_v = next(it)
    ctmp = next(it)
    invb = next(it)
    nbv = next(it)
    csem = next(it)

    c = lax.axis_index("c")
    s = lax.axis_index("s")
    iota = lax.iota(jnp.int32, 16)
    lo = s * RPT
    hi = lo + RPT
    pltpu.sync_copy(ones_h, ones_v)

    for r in range(4):
        @pl.when(c == r // 2)
        def _():
            base = (r * NSUB + s) * CAPB * LB

            @pl.loop(0, (LB + 16) // 16)
            def _(j):
                cbs[pl.ds(j * 16, 16)] = jnp.zeros((16,), jnp.int32)
                cbd[pl.ds(j * 16, 16)] = jnp.zeros((16,), jnp.int32) + TRASH

            def chunk_body(ci, carry):
                off0, tot0 = carry
                pltpu.sync_copy(srcs[r].at[pl.ds(ci * SCH, SCH)], sbuf)
                pltpu.sync_copy(dsts[r].at[pl.ds(ci * SCH, SCH)], dbuf)

                def grp_body(k, carry2):
                    off, tot = carry2
                    sv = sbuf[pl.ds(k * 16, 16)]
                    dv = dbuf[pl.ds(k * 16, 16)]
                    m = (dv >= lo) & (dv < hi)
                    mi = m.astype(jnp.int32)
                    p = mi
                    for sh in (1, 2, 4, 8):
                        shifted = p.at[jnp.maximum(iota - sh, 0)].get(
                            mode="promise_in_bounds")
                        p = p + jnp.where(iota >= sh, shifted, 0)
                    pos = off + p - mi
                    plsc.store_scatter(cbs, [pos], sv, mask=m)
                    plsc.store_scatter(cbd, [pos], dv, mask=m)
                    pc = jnp.max(plsc.all_reduce_population_count(m))
                    off2 = off + pc
                    full = off2 >= LB

                    @pl.when(full)
                    def _():
                        pltpu.sync_copy(cbs.at[pl.ds(0, LB)],
                                        slist.at[pl.ds(base + tot * LB, LB)])
                        pltpu.sync_copy(cbd.at[pl.ds(0, LB)],
                                        dlist.at[pl.ds(base + tot * LB, LB)])
                        vs = cbs[pl.ds(LB, 16)]
                        vd = cbd[pl.ds(LB, 16)]

                        @pl.loop(0, (LB + 16) // 16)
                        def _(j):
                            cbs[pl.ds(j * 16, 16)] = jnp.zeros((16,), jnp.int32)
                            cbd[pl.ds(j * 16, 16)] = (
                                jnp.zeros((16,), jnp.int32) + TRASH)
                        rem = off2 - LB
                        cbs[pl.ds(0, 16)] = jnp.where(iota < rem, vs, 0)
                        cbd[pl.ds(0, 16)] = jnp.where(iota < rem, vd, TRASH)

                    off3 = jnp.where(full, off2 - LB, off2)
                    tot3 = tot + full.astype(jnp.int32)
                    return (off3, tot3)

                return pl.loop(0, SCH // 16, init_carry=(off0, tot0))(grp_body)

            off, tot = pl.loop(
                0, E // SCH,
                init_carry=(jnp.int32(0), jnp.int32(0)))(chunk_body)

            @pl.when(off > 0)
            def _():
                pltpu.sync_copy(cbs.at[pl.ds(0, LB)],
                                slist.at[pl.ds(base + tot * LB, LB)])
                pltpu.sync_copy(cbd.at[pl.ds(0, LB)],
                                dlist.at[pl.ds(base + tot * LB, LB)])
            nb = tot + (off > 0).astype(jnp.int32)

            nbv[pl.ds(0, 16)] = jnp.zeros((16,), jnp.int32) + nb
            pltpu.sync_copy(nbv, nb_out.at[pl.ds((r * NSUB + s) * 16, 16)])

            @pl.loop(0, NFCH)
            def _(k):
                pltpu.sync_copy(z16_h, cacc.at[pl.ds(s * RPT + k * FCH, FCH)])

            @pl.loop(0, nb)
            def _(b):
                pltpu.sync_copy(dlist.at[pl.ds(base + b * LB, LB)], dbuf2)
                cps = [pltpu.async_copy(ones_v,
                                        cacc.at[dbuf2.at[pl.ds(j * CH, CH)]],
                                        csem, add=True)
                       for j in range(LB // CH)]
                for cp in cps:
                    cp.wait()

            @pl.loop(0, NFCH)
            def _(k):
                row0 = s * RPT + k * FCH
                pltpu.sync_copy(cacc.at[pl.ds(row0, FCH)], ctmp)

                @pl.loop(0, FCH)
                def _(rr):
                    v = ctmp[rr, pl.ds(0, 16)]
                    invb[pl.ds(rr * 16, 16)] = 1.0 / jnp.maximum(v, 1.0)

                pltpu.sync_copy(invb,
                                inv_out.at[pl.ds((r * NP + row0) * 16,
                                                 FCH * 16)])


def _sc_prep(srcs, dsts):
    SZ = 4 * NSUB * CAPB * LB
    fn = pl.kernel(
        _sc_prep_body,
        out_type=[
            jax.ShapeDtypeStruct((SZ,), jnp.int32),
            jax.ShapeDtypeStruct((SZ,), jnp.int32),
            jax.ShapeDtypeStruct((4 * NSUB * 16,), jnp.int32),
            jax.ShapeDtypeStruct((4 * NP * 16,), F32),
        ],
        mesh=_MESH,
        scratch_types=[
            pltpu.VMEM_SHARED((NP, 16), F32),
            pltpu.VMEM((SCH,), jnp.int32),
            pltpu.VMEM((SCH,), jnp.int32),
            pltpu.VMEM((LB + 16,), jnp.int32),
            pltpu.VMEM((LB + 16,), jnp.int32),
            pltpu.VMEM((LB,), jnp.int32),
            pltpu.VMEM((CH, 16), F32),
            pltpu.VMEM((FCH, 16), F32),
            pltpu.VMEM((FCH * 16,), F32),
            pltpu.VMEM((16,), jnp.int32),
            pltpu.SemaphoreType.DMA,
        ],
        compiler_params=_SC_PARAMS,
    )
    return fn(*srcs, *dsts, jnp.ones((CH, 16), F32),
              jnp.zeros((FCH, 16), F32))


def _sc_agg_body(*refs):
    it = iter(refs)
    slist = next(it)
    dlist = next(it)
    nb_in = next(it)
    inv_in = next(it)
    tbls = [next(it) for _ in range(4)]
    zrow = next(it)
    outs = [next(it) for _ in range(4)]
    acc = next(it)
    sidx = next(it)
    didx = next(it)
    rows = [next(it) for _ in range(KB)]
    ftmp = next(it)
    invb = next(it)
    nbv = next(it)
    gsem = next(it)
    ssem = next(it)

    c = lax.axis_index("c")
    s = lax.axis_index("s")
    off = c * N

    for r in range(4):
        tbl, out = tbls[r], outs[r]
        base = (r * NSUB + s) * CAPB * LB

        @pl.loop(0, RPT // 64)
        def _(k):
            pltpu.sync_copy(zrow, acc.at[pl.ds(s * RPT + k * 64, 64)])

        pltpu.sync_copy(nb_in.at[pl.ds((r * NSUB + s) * 16, 16)], nbv)
        nb = jnp.max(nbv[pl.ds(0, 16)])

        @pl.loop(0, nb)
        def _(b):
            pltpu.sync_copy(slist.at[pl.ds(base + b * LB, LB)], sidx)
            pltpu.sync_copy(dlist.at[pl.ds(base + b * LB, LB)], didx)

            @pl.loop(0, LB // 16)
            def _(k):
                sl = pl.ds(k * 16, 16)
                sidx[sl] = sidx[sl] + off

            @pl.loop(0, LB // (CH * KB))
            def _(g):
                gbase = g * KB
                gcps = [pltpu.async_copy(
                            tbl.at[sidx.at[pl.ds((gbase + j) * CH, CH)]],
                            rows[j], gsem)
                        for j in range(KB)]
                for cp in gcps:
                    cp.wait()
                scps = [pltpu.async_copy(
                            rows[j],
                            acc.at[didx.at[pl.ds((gbase + j) * CH, CH)]],
                            ssem, add=True)
                        for j in range(KB)]
                for cp in scps:
                    cp.wait()

        @pl.loop(0, NFCH)
        def _(k):
            row0 = s * RPT + k * FCH
            pltpu.sync_copy(inv_in.at[pl.ds((r * NP + row0) * 16, FCH * 16)],
                            invb)
            pltpu.sync_copy(acc.at[pl.ds(row0, FCH)], ftmp)

            @pl.loop(0, FCH)
            def _(rr):
                iv = invb[pl.ds(rr * 16, 16)]
                for j in range(8):
                    sl2 = pl.ds(j * 16, 16)
                    ftmp[rr, sl2] = ftmp[rr, sl2] * iv

            pltpu.sync_copy(ftmp, out.at[pl.ds(c * NP + row0, FCH)])


def _sc_aggregate(tables, slist, dlist, nblocks, inv):
    out_type = [jax.ShapeDtypeStruct((2 * NP, 128), F32) for _ in range(4)]
    scratch = [
        pltpu.VMEM_SHARED((NP, 128), F32),
        pltpu.VMEM((LB,), jnp.int32),
        pltpu.VMEM((LB,), jnp.int32),
        *[pltpu.VMEM((CH, 128), F32) for _ in range(KB)],
        pltpu.VMEM((FCH, 128), F32),
        pltpu.VMEM((FCH * 16,), F32),
        pltpu.VMEM((16,), jnp.int32),
        pltpu.SemaphoreType.DMA,
        pltpu.SemaphoreType.DMA,
    ]
    fn = pl.kernel(_sc_agg_body, out_type=out_type, mesh=_MESH,
                   scratch_types=scratch, compiler_params=_SC_PARAMS)
    return fn(slist, dlist, nblocks, inv, *tables,
              jnp.zeros((64, 128), F32))


def kernel(x_sub, x_module, params, node_id_sub, node_id_bay, node_id_module,
           edge_index_sub_bay, edge_index_bay_module):
    p = params
    sb0, sb1 = edge_index_sub_bay[0], edge_index_sub_bay[1]
    bm0, bm1 = edge_index_bay_module[0], edge_index_bay_module[1]

    srcs_f = [sb0, bm1, sb1, bm0]
    dsts_f = [sb1, bm0, sb0, bm1]

    def cw_sub(l):
        return jnp.concatenate([p['l%d_sb_Wl' % l], p['l%d_bs_Wr' % l]], axis=1)

    def cw_bay(l):
        return jnp.concatenate([p['l%d_bs_Wl' % l], p['l%d_bm_Wl' % l],
                                p['l%d_sb_Wr' % l] + p['l%d_mb_Wr' % l]], axis=1)

    def cw_mod(l):
        return jnp.concatenate([p['l%d_mb_Wl' % l], p['l%d_bm_Wr' % l]], axis=1)

    tbl_sb, self_sub = _enc_proj(x_sub, p['W_lin_sub'], p['b_lin_sub'],
                                 p['emb_sub'], cw_sub(1), 1)
    tbl_bs, tbl_bm, self_bay = _plain_proj(p['emb_bay'], cw_bay(1), 2)
    tbl_mb, self_mod = _enc_proj(x_module, p['W_lin_module'], p['b_lin_module'],
                                 p['emb_module'], cw_mod(1), 1)

    slist, dlist, nblocks, inv_flat = _sc_prep(srcs_f, dsts_f)
    for l in (1, 2, 3):
        flat = [t.reshape(2 * N, 128) for t in (tbl_sb, tbl_mb, tbl_bs, tbl_bm)]
        res = _sc_aggregate(flat, slist, dlist, nblocks, inv_flat)
        a_sb, a_mb, a_bs, a_bm = (x.reshape(2, NP, 128)[:, :N, :]
                                  for x in res)
        bl_sub = p['l%d_bs_bl' % l]
        bl_bay = p['l%d_sb_bl' % l] + p['l%d_mb_bl' % l]
        bl_mod = p['l%d_bm_bl' % l]
        if l < 3:
            tbl_sb, self_sub = _combine_proj([a_bs], self_sub, bl_sub,
                                             cw_sub(l + 1), 1)
            tbl_bs, tbl_bm, self_bay = _combine_proj([a_sb, a_mb], self_bay,
                                                     bl_bay, cw_bay(l + 1), 2)
            tbl_mb, self_mod = _combine_proj([a_bm], self_mod, bl_mod,
                                             cw_mod(l + 1), 1)
        else:
            out_sub = _combine_final([a_bs], self_sub, bl_sub)
            out_bay = _combine_final([a_sb, a_mb], self_bay, bl_bay)
            out_mod = _combine_final([a_bm], self_mod, bl_mod)

    return (out_sub, out_bay, out_mod)
